# Initial kernel scaffold; baseline (speedup 1.0000x reference)
import jax
import jax.numpy as jnp
from jax.experimental import pallas as pl


def kernel(heads, relations, tails, entity_table, relation_table):
    raise NotImplementedError("write your pallas kernel here")


# speedup vs baseline: 1.2414x; 1.2414x over previous
import functools

import jax
import jax.numpy as jnp
from jax import lax
from jax.experimental import pallas as pl
from jax.experimental.pallas import tpu as pltpu
from jax.experimental.pallas import tpu_sc as plsc

_B = 16384
_D = 128
_HD = _D // 2
_NC, _NS = 2, 16
_NW = _NC * _NS
_BPW = _B // _NW
_C = 128
_NCHUNK = _BPW // _C


@functools.lru_cache(maxsize=None)
def _make_sc_kernel(interpret=False):
    mesh = plsc.VectorSubcoreMesh(
        core_axis_name="c", subcore_axis_name="s",
        num_cores=_NC, num_subcores=_NS,
    )

    @functools.partial(
        pl.kernel,
        out_type=jax.ShapeDtypeStruct((_B,), jnp.float32),
        mesh=mesh,
        scratch_types=[
            pltpu.VMEM((_C,), jnp.int32),
            pltpu.VMEM((_C,), jnp.int32),
            pltpu.VMEM((_C,), jnp.int32),
            pltpu.VMEM((_C, _D), jnp.float32),
            pltpu.VMEM((_C, _D), jnp.float32),
            pltpu.VMEM((_C, _D), jnp.float32),
            pltpu.VMEM((_BPW,), jnp.float32),
            pltpu.SemaphoreType.DMA,
        ],
        compiler_params=pltpu.CompilerParams(needs_layout_passes=False),
        interpret=interpret,
    )
    def sc_kernel(heads_hbm, rel_hbm, tails_hbm, ent_hbm, rtab_hbm, out_hbm,
                  hidx, ridx, tidx, hrows, rrows, trows, outv, sem):
        wid = lax.axis_index("s") * _NC + lax.axis_index("c")
        base = wid * _BPW

        def chunk_body(c, carry):
            cb = c * _C
            pltpu.sync_copy(heads_hbm.at[pl.ds(base + cb, _C)], hidx)
            pltpu.sync_copy(rel_hbm.at[pl.ds(base + cb, _C)], ridx)
            pltpu.sync_copy(tails_hbm.at[pl.ds(base + cb, _C)], tidx)
            cph = pltpu.async_copy(ent_hbm.at[hidx], hrows, sem)
            cpr = pltpu.async_copy(rtab_hbm.at[ridx], rrows, sem)
            cpt = pltpu.async_copy(ent_hbm.at[tidx], trows, sem)
            cph.wait()
            cpr.wait()
            cpt.wait()

            def group_body(g, gcarry):
                row0 = g * 16
                rows = row0 + lax.iota(jnp.int32, 16)

                def dd_body(dd, acc):
                    cre = jnp.full((16,), dd, jnp.int32)
                    cim = cre + _HD
                    hre = plsc.load_gather(hrows, [rows, cre])
                    him = plsc.load_gather(hrows, [rows, cim])
                    rre = plsc.load_gather(rrows, [rows, cre])
                    rim = plsc.load_gather(rrows, [rows, cim])
                    tre = plsc.load_gather(trows, [rows, cre])
                    tim = plsc.load_gather(trows, [rows, cim])
                    return acc + hre * (rre * tre + rim * tim) \
                               + him * (rre * tim - rim * tre)

                acc = lax.fori_loop(0, _HD, dd_body,
                                    jnp.zeros((16,), jnp.float32), unroll=8)
                outv[pl.ds(cb + row0, 16)] = acc
                return gcarry

            lax.fori_loop(0, _C // 16, group_body, 0)
            return carry

        lax.fori_loop(0, _NCHUNK, chunk_body, 0)
        pltpu.sync_copy(outv, out_hbm.at[pl.ds(base, _BPW)])

    return sc_kernel


def kernel(heads, relations, tails, entity_table, relation_table):
    out = _make_sc_kernel()(
        heads.astype(jnp.int32),
        relations.astype(jnp.int32),
        tails.astype(jnp.int32),
        entity_table,
        relation_table,
    )
    return out.reshape(_B, 1)


# speedup vs baseline: 1.3585x; 1.0943x over previous
import functools

import jax
import jax.numpy as jnp
from jax import lax
from jax.experimental import pallas as pl
from jax.experimental.pallas import tpu as pltpu
from jax.experimental.pallas import tpu_sc as plsc

_B = 16384
_D = 128
_HD = _D // 2
_NC, _NS = 2, 16
_NW = _NC * _NS
_BPW = _B // _NW
_C = 128
_NCHUNK = _BPW // _C


@functools.lru_cache(maxsize=None)
def _make_sc_kernel():
    mesh = plsc.VectorSubcoreMesh(
        core_axis_name="c", subcore_axis_name="s",
        num_cores=_NC, num_subcores=_NS,
    )

    @functools.partial(
        pl.kernel,
        out_type=jax.ShapeDtypeStruct((_B,), jnp.float32),
        mesh=mesh,
        scratch_types=[
            pltpu.VMEM((_BPW,), jnp.int32),
            pltpu.VMEM((_BPW,), jnp.int32),
            pltpu.VMEM((_BPW,), jnp.int32),
            pltpu.VMEM((2, _C, _D), jnp.float32),
            pltpu.VMEM((2, _C, _D), jnp.float32),
            pltpu.VMEM((2, _C, _D), jnp.float32),
            pltpu.VMEM((_BPW,), jnp.float32),
            pltpu.SemaphoreType.DMA,
            pltpu.SemaphoreType.DMA,
        ],
        compiler_params=pltpu.CompilerParams(needs_layout_passes=False),
    )
    def sc_kernel(heads_hbm, rel_hbm, tails_hbm, ent_hbm, rtab_hbm, out_hbm,
                  hidx, ridx, tidx, hrows, rrows, trows, outv, sem0, sem1):
        wid = lax.axis_index("s") * _NC + lax.axis_index("c")
        base = wid * _BPW

        ci0 = pltpu.async_copy(heads_hbm.at[pl.ds(base, _BPW)], hidx, sem0)
        ci1 = pltpu.async_copy(rel_hbm.at[pl.ds(base, _BPW)], ridx, sem0)
        ci2 = pltpu.async_copy(tails_hbm.at[pl.ds(base, _BPW)], tidx, sem0)
        ci0.wait()
        ci1.wait()
        ci2.wait()

        sems = (sem0, sem1)

        def issue(c):
            slot = c % 2
            cb = c * _C
            sem = sems[slot]
            return (
                pltpu.async_copy(
                    ent_hbm.at[hidx.at[pl.ds(cb, _C)]], hrows.at[slot], sem),
                pltpu.async_copy(
                    rtab_hbm.at[ridx.at[pl.ds(cb, _C)]], rrows.at[slot], sem),
                pltpu.async_copy(
                    ent_hbm.at[tidx.at[pl.ds(cb, _C)]], trows.at[slot], sem),
            )

        def compute(c):
            slot = c % 2
            cb = c * _C
            hb, rb, tb = hrows.at[slot], rrows.at[slot], trows.at[slot]

            def group_body(g, gcarry):
                row0 = g * 16
                rows = row0 + lax.iota(jnp.int32, 16)

                def dd_body(dd, acc):
                    cre = jnp.full((16,), dd, jnp.int32)
                    cim = cre + _HD
                    hre = plsc.load_gather(hb, [rows, cre])
                    him = plsc.load_gather(hb, [rows, cim])
                    rre = plsc.load_gather(rb, [rows, cre])
                    rim = plsc.load_gather(rb, [rows, cim])
                    tre = plsc.load_gather(tb, [rows, cre])
                    tim = plsc.load_gather(tb, [rows, cim])
                    return acc + hre * (rre * tre + rim * tim) \
                               + him * (rre * tim - rim * tre)

                acc = lax.fori_loop(0, _HD, dd_body,
                                    jnp.zeros((16,), jnp.float32), unroll=8)
                outv[pl.ds(cb + row0, 16)] = acc
                return gcarry

            lax.fori_loop(0, _C // 16, group_body, 0)

        pend = issue(0)
        for c in range(_NCHUNK):
            nxt = issue(c + 1) if c + 1 < _NCHUNK else None
            for p in pend:
                p.wait()
            compute(c)
            pend = nxt

        pltpu.sync_copy(outv, out_hbm.at[pl.ds(base, _BPW)])

    return sc_kernel


def kernel(heads, relations, tails, entity_table, relation_table):
    out = _make_sc_kernel()(
        heads.astype(jnp.int32),
        relations.astype(jnp.int32),
        tails.astype(jnp.int32),
        entity_table,
        relation_table,
    )
    return out.reshape(_B, 1)


# speedup vs baseline: 1.3603x; 1.0013x over previous
import functools

import jax
import jax.numpy as jnp
from jax import lax
from jax.experimental import pallas as pl
from jax.experimental.pallas import tpu as pltpu
from jax.experimental.pallas import tpu_sc as plsc

_B = 16384
_D = 128
_HD = _D // 2
_NC, _NS = 2, 16
_NW = _NC * _NS
_BPW = _B // _NW
_C = 128
_NCHUNK = _BPW // _C


@functools.lru_cache(maxsize=None)
def _make_sc_kernel():
    mesh = plsc.VectorSubcoreMesh(
        core_axis_name="c", subcore_axis_name="s",
        num_cores=_NC, num_subcores=_NS,
    )

    @functools.partial(
        pl.kernel,
        out_type=jax.ShapeDtypeStruct((_B,), jnp.float32),
        mesh=mesh,
        scratch_types=[
            pltpu.VMEM((_BPW,), jnp.int32),
            pltpu.VMEM((_BPW,), jnp.int32),
            pltpu.VMEM((_BPW,), jnp.int32),
            pltpu.VMEM((2, _C, _D), jnp.float32),
            pltpu.VMEM((2, _C, _D), jnp.float32),
            pltpu.VMEM((2, _C, _D), jnp.float32),
            pltpu.VMEM((_BPW,), jnp.float32),
            pltpu.SemaphoreType.DMA,
            pltpu.SemaphoreType.DMA,
        ],
        compiler_params=pltpu.CompilerParams(needs_layout_passes=False),
    )
    def sc_kernel(heads_hbm, rel_hbm, tails_hbm, ent_hbm, rtab_hbm, out_hbm,
                  hidx, ridx, tidx, hrows, rrows, trows, outv, sem0, sem1):
        wid = lax.axis_index("s") * _NC + lax.axis_index("c")
        base = wid * _BPW

        ci0 = pltpu.async_copy(heads_hbm.at[pl.ds(base, _BPW)], hidx, sem0)
        ci1 = pltpu.async_copy(rel_hbm.at[pl.ds(base, _BPW)], ridx, sem0)
        ci2 = pltpu.async_copy(tails_hbm.at[pl.ds(base, _BPW)], tidx, sem0)
        ci0.wait()
        ci1.wait()
        ci2.wait()

        sems = (sem0, sem1)

        def issue(c):
            slot = c % 2
            cb = c * _C
            sem = sems[slot]
            return (
                pltpu.async_copy(
                    ent_hbm.at[hidx.at[pl.ds(cb, _C)]], hrows.at[slot], sem),
                pltpu.async_copy(
                    rtab_hbm.at[ridx.at[pl.ds(cb, _C)]], rrows.at[slot], sem),
                pltpu.async_copy(
                    ent_hbm.at[tidx.at[pl.ds(cb, _C)]], trows.at[slot], sem),
            )

        def compute(c):
            slot = c % 2
            cb = c * _C
            hb, rb, tb = hrows.at[slot], rrows.at[slot], trows.at[slot]

            def group_body(g, gcarry):
                row0 = g * 16
                rows = row0 + lax.iota(jnp.int32, 16)

                @plsc.parallel_loop(0, _HD, unroll=8,
                                    carry=jnp.zeros((16,), jnp.float32))
                def dd_body(dd, acc):
                    cre = jnp.full((16,), dd, jnp.int32)
                    cim = cre + _HD
                    hre = plsc.load_gather(hb, [rows, cre])
                    him = plsc.load_gather(hb, [rows, cim])
                    rre = plsc.load_gather(rb, [rows, cre])
                    rim = plsc.load_gather(rb, [rows, cim])
                    tre = plsc.load_gather(tb, [rows, cre])
                    tim = plsc.load_gather(tb, [rows, cim])
                    return acc + hre * (rre * tre + rim * tim) \
                               + him * (rre * tim - rim * tre)

                acc = dd_body
                outv[pl.ds(cb + row0, 16)] = acc
                return gcarry

            lax.fori_loop(0, _C // 16, group_body, 0)

        pend = issue(0)
        for c in range(_NCHUNK):
            nxt = issue(c + 1) if c + 1 < _NCHUNK else None
            for p in pend:
                p.wait()
            compute(c)
            pend = nxt

        pltpu.sync_copy(outv, out_hbm.at[pl.ds(base, _BPW)])

    return sc_kernel


def kernel(heads, relations, tails, entity_table, relation_table):
    out = _make_sc_kernel()(
        heads.astype(jnp.int32),
        relations.astype(jnp.int32),
        tails.astype(jnp.int32),
        entity_table,
        relation_table,
    )
    return out.reshape(_B, 1)


# speedup vs baseline: 4.1760x; 3.0700x over previous
import functools

import jax
import jax.numpy as jnp
from jax import lax
from jax.experimental import pallas as pl
from jax.experimental.pallas import tpu as pltpu
from jax.experimental.pallas import tpu_sc as plsc

_B = 16384
_D = 128
_HD = _D // 2
_NC, _NS = 2, 16
_NW = _NC * _NS
_BPW = _B // _NW
_C = 128
_NCHUNK = _BPW // _C


@functools.lru_cache(maxsize=None)
def _make_sc_kernel():
    mesh = plsc.VectorSubcoreMesh(
        core_axis_name="c", subcore_axis_name="s",
        num_cores=_NC, num_subcores=_NS,
    )

    @functools.partial(
        pl.kernel,
        out_type=jax.ShapeDtypeStruct((_B,), jnp.float32),
        mesh=mesh,
        scratch_types=[
            pltpu.VMEM((_BPW,), jnp.int32),
            pltpu.VMEM((_BPW,), jnp.int32),
            pltpu.VMEM((_BPW,), jnp.int32),
            pltpu.VMEM((2, _C, _D), jnp.float32),
            pltpu.VMEM((2, _C, _D), jnp.float32),
            pltpu.VMEM((2, _C, _D), jnp.float32),
            pltpu.VMEM((_BPW,), jnp.float32),
            pltpu.SemaphoreType.DMA,
            pltpu.SemaphoreType.DMA,
        ],
        compiler_params=pltpu.CompilerParams(needs_layout_passes=False),
    )
    def sc_kernel(heads_hbm, rel_hbm, tails_hbm, ent_hbm, rtab_hbm, out_hbm,
                  hidx, ridx, tidx, hrows, rrows, trows, outv, sem0, sem1):
        wid = lax.axis_index("s") * _NC + lax.axis_index("c")
        base = wid * _BPW

        ci0 = pltpu.async_copy(heads_hbm.at[pl.ds(base, _BPW)], hidx, sem0)
        ci1 = pltpu.async_copy(rel_hbm.at[pl.ds(base, _BPW)], ridx, sem0)
        ci2 = pltpu.async_copy(tails_hbm.at[pl.ds(base, _BPW)], tidx, sem0)
        ci0.wait()
        ci1.wait()
        ci2.wait()

        sems = (sem0, sem1)

        def issue(c):
            slot = c % 2
            cb = c * _C
            sem = sems[slot]
            return (
                pltpu.async_copy(
                    ent_hbm.at[hidx.at[pl.ds(cb, _C)]], hrows.at[slot], sem),
                pltpu.async_copy(
                    rtab_hbm.at[ridx.at[pl.ds(cb, _C)]], rrows.at[slot], sem),
                pltpu.async_copy(
                    ent_hbm.at[tidx.at[pl.ds(cb, _C)]], trows.at[slot], sem),
            )

        lane15 = lax.iota(jnp.int32, 16) == 15

        def compute(c):
            slot = c % 2
            cb = c * _C
            hb, rb, tb = hrows.at[slot], rrows.at[slot], trows.at[slot]

            @plsc.parallel_loop(0, _C, unroll=4, carry=jnp.int32(0))
            def row_body(i, rcarry):
                parts = []
                for j in range(_HD // 16):
                    re_s = pl.ds(j * 16, 16)
                    im_s = pl.ds(_HD + j * 16, 16)
                    hre = hb[i, re_s]
                    him = hb[i, im_s]
                    rre = rb[i, re_s]
                    rim = rb[i, im_s]
                    tre = tb[i, re_s]
                    tim = tb[i, im_s]
                    parts.append(hre * (rre * tre + rim * tim)
                                 + him * (rre * tim - rim * tre))
                acc = (parts[0] + parts[1]) + (parts[2] + parts[3])
                total = plsc.cumsum(acc)
                plsc.store_scatter(outv, [jnp.full((16,), cb + i, jnp.int32)],
                                   total, mask=lane15)
                return rcarry

        pend = issue(0)
        for c in range(_NCHUNK):
            nxt = issue(c + 1) if c + 1 < _NCHUNK else None
            for p in pend:
                p.wait()
            compute(c)
            pend = nxt

        pltpu.sync_copy(outv, out_hbm.at[pl.ds(base, _BPW)])

    return sc_kernel


def kernel(heads, relations, tails, entity_table, relation_table):
    out = _make_sc_kernel()(
        heads.astype(jnp.int32),
        relations.astype(jnp.int32),
        tails.astype(jnp.int32),
        entity_table,
        relation_table,
    )
    return out.reshape(_B, 1)
